# baseline (speedup 1.0000x reference)
import jax
import jax.numpy as jnp
from jax.experimental import pallas as pl
from jax.experimental.pallas import tpu as pltpu

_BUCKET = 128


def _router_kernel(q_ref, k_ref, idx_ref, w_ref):
    t, dh = q_ref.shape[2], q_ref.shape[3]
    nb = t // _BUCKET
    scale = dh ** -0.5

    sq = jnp.concatenate(
        [jnp.mean(q_ref[0, 0, u * _BUCKET:(u + 1) * _BUCKET, :], axis=0,
                  keepdims=True) for u in range(nb)], axis=0)
    sk = jnp.concatenate(
        [jnp.mean(k_ref[0, 0, u * _BUCKET:(u + 1) * _BUCKET, :], axis=0,
                  keepdims=True) for u in range(nb)], axis=0)
    r = jax.lax.dot_general(sq, sk, (((1,), (1,)), ((), ())),
                            preferred_element_type=jnp.float32) * scale
    rmax = jnp.max(r, axis=1, keepdims=True)
    w = 1.0 / jnp.sum(jnp.exp(r - rmax), axis=1, keepdims=True)
    iota = jax.lax.broadcasted_iota(jnp.int32, (nb, nb), 1)
    idx = jnp.min(jnp.where(r >= rmax, iota, nb), axis=1, keepdims=True)
    idx_ref[0] = idx
    w_ref[0] = w


def _attn_kernel(idx_sref, w_sref, q_ref, k_ref, v_ref, o_ref):
    t, dh = q_ref.shape[2], q_ref.shape[3]
    nb = t // _BUCKET
    scale = dh ** -0.5
    h = pl.num_programs(1)
    i = pl.program_id(0) * h + pl.program_id(1)

    for u in range(nb):
        idx_u = idx_sref[i, u]
        w_u = w_sref[i, u]

        qb = q_ref[0, 0, u * _BUCKET:(u + 1) * _BUCKET, :] * scale
        kl = k_ref[0, 0, u * _BUCKET:(u + 1) * _BUCKET, :]
        vl = v_ref[0, 0, u * _BUCKET:(u + 1) * _BUCKET, :]
        kg = k_ref[0, 0, pl.ds(idx_u * _BUCKET, _BUCKET), :]
        vg = v_ref[0, 0, pl.ds(idx_u * _BUCKET, _BUCKET), :]

        kcat = jnp.concatenate([kg * w_u, kl], axis=0)
        vcat = jnp.concatenate([vg * w_u, vl], axis=0)
        s = jax.lax.dot_general(qb, kcat, (((1,), (1,)), ((), ())),
                                preferred_element_type=jnp.float32)
        smax = jnp.max(s, axis=1, keepdims=True)
        p = jnp.exp(s - smax)
        den = jnp.sum(p, axis=1, keepdims=True)
        o = jax.lax.dot_general(p, vcat, (((1,), (0,)), ((), ())),
                                preferred_element_type=jnp.float32)
        o_ref[0, 0, u * _BUCKET:(u + 1) * _BUCKET, :] = o / den


def kernel(q, k, v):
    b, h, t, dh = q.shape
    bh = b * h
    nb = t // _BUCKET

    idx3, w3 = pl.pallas_call(
        _router_kernel,
        grid=(b, h),
        in_specs=[
            pl.BlockSpec((1, 1, t, dh), lambda i, j: (i, j, 0, 0)),
            pl.BlockSpec((1, 1, t, dh), lambda i, j: (i, j, 0, 0)),
        ],
        out_specs=[
            pl.BlockSpec((1, nb, 1), lambda i, j: (i * h + j, 0, 0)),
            pl.BlockSpec((1, nb, 1), lambda i, j: (i * h + j, 0, 0)),
        ],
        out_shape=[
            jax.ShapeDtypeStruct((bh, nb, 1), jnp.int32),
            jax.ShapeDtypeStruct((bh, nb, 1), jnp.float32),
        ],
        compiler_params=pltpu.CompilerParams(
            dimension_semantics=("arbitrary", "arbitrary")),
    )(q, k)
    idx = idx3.reshape(bh, nb)
    w = w3.reshape(bh, nb)

    grid_spec = pltpu.PrefetchScalarGridSpec(
        num_scalar_prefetch=2,
        grid=(b, h),
        in_specs=[
            pl.BlockSpec((1, 1, t, dh), lambda i, j, *_: (i, j, 0, 0)),
            pl.BlockSpec((1, 1, t, dh), lambda i, j, *_: (i, j, 0, 0)),
            pl.BlockSpec((1, 1, t, dh), lambda i, j, *_: (i, j, 0, 0)),
        ],
        out_specs=pl.BlockSpec((1, 1, t, dh), lambda i, j, *_: (i, j, 0, 0)),
    )
    out = pl.pallas_call(
        _attn_kernel,
        grid_spec=grid_spec,
        out_shape=jax.ShapeDtypeStruct((b, h, t, dh), q.dtype),
        compiler_params=pltpu.CompilerParams(
            dimension_semantics=("arbitrary", "arbitrary")),
    )(idx, w, q, k, v)
    return out


# speedup vs baseline: 1.0010x; 1.0010x over previous
import jax
import jax.numpy as jnp
from jax.experimental import pallas as pl
from jax.experimental.pallas import tpu as pltpu

_BUCKET = 128


def _router_kernel(q_ref, k_ref, idx_ref, w_ref):
    t, dh = q_ref.shape[2], q_ref.shape[3]
    nb = t // _BUCKET
    scale = dh ** -0.5

    sq = jnp.concatenate(
        [jnp.mean(q_ref[0, 0, u * _BUCKET:(u + 1) * _BUCKET, :], axis=0,
                  keepdims=True) for u in range(nb)], axis=0)
    sk = jnp.concatenate(
        [jnp.mean(k_ref[0, 0, u * _BUCKET:(u + 1) * _BUCKET, :], axis=0,
                  keepdims=True) for u in range(nb)], axis=0)
    r = jax.lax.dot_general(sq, sk, (((1,), (1,)), ((), ())),
                            preferred_element_type=jnp.float32) * scale
    rmax = jnp.max(r, axis=1, keepdims=True)
    w = 1.0 / jnp.sum(jnp.exp(r - rmax), axis=1, keepdims=True)
    iota = jax.lax.broadcasted_iota(jnp.int32, (nb, nb), 1)
    idx = jnp.min(jnp.where(r >= rmax, iota, nb), axis=1, keepdims=True)
    idx_ref[0] = idx
    w_ref[0] = w


def _attn_kernel(idx_sref, w_sref, q_ref, k_ref, v_ref, o_ref):
    t, dh = q_ref.shape[2], q_ref.shape[3]
    nb = t // _BUCKET
    scale = dh ** -0.5
    i = pl.program_id(0)

    for u in range(nb):
        idx_u = idx_sref[i, u]
        w_u = w_sref[i, u]

        qb = q_ref[0, 0, u * _BUCKET:(u + 1) * _BUCKET, :] * scale
        kl = k_ref[0, 0, u * _BUCKET:(u + 1) * _BUCKET, :]
        vl = v_ref[0, 0, u * _BUCKET:(u + 1) * _BUCKET, :]
        kg = k_ref[0, 0, pl.ds(idx_u * _BUCKET, _BUCKET), :]
        vg = v_ref[0, 0, pl.ds(idx_u * _BUCKET, _BUCKET), :]

        kcat = jnp.concatenate([kg * w_u, kl], axis=0)
        vcat = jnp.concatenate([vg * w_u, vl], axis=0)
        s = jax.lax.dot_general(qb, kcat, (((1,), (1,)), ((), ())),
                                preferred_element_type=jnp.float32)
        smax = jnp.max(s, axis=1, keepdims=True)
        p = jnp.exp(s - smax)
        den = jnp.sum(p, axis=1, keepdims=True)
        o = jax.lax.dot_general(p, vcat, (((1,), (0,)), ((), ())),
                                preferred_element_type=jnp.float32)
        o_ref[0, 0, u * _BUCKET:(u + 1) * _BUCKET, :] = o / den


def kernel(q, k, v):
    b, h, t, dh = q.shape
    bh = b * h
    nb = t // _BUCKET

    idx3, w3 = pl.pallas_call(
        _router_kernel,
        grid=(bh,),
        in_specs=[
            pl.BlockSpec((1, 1, t, dh), lambda i: (i // h, i % h, 0, 0)),
            pl.BlockSpec((1, 1, t, dh), lambda i: (i // h, i % h, 0, 0)),
        ],
        out_specs=[
            pl.BlockSpec((1, nb, 1), lambda i: (i, 0, 0)),
            pl.BlockSpec((1, nb, 1), lambda i: (i, 0, 0)),
        ],
        out_shape=[
            jax.ShapeDtypeStruct((bh, nb, 1), jnp.int32),
            jax.ShapeDtypeStruct((bh, nb, 1), jnp.float32),
        ],
        compiler_params=pltpu.CompilerParams(
            dimension_semantics=("arbitrary",)),
    )(q, k)
    idx = idx3.reshape(bh, nb)
    w = w3.reshape(bh, nb)

    grid_spec = pltpu.PrefetchScalarGridSpec(
        num_scalar_prefetch=2,
        grid=(bh,),
        in_specs=[
            pl.BlockSpec((1, 1, t, dh), lambda i, *_: (i // h, i % h, 0, 0)),
            pl.BlockSpec((1, 1, t, dh), lambda i, *_: (i // h, i % h, 0, 0)),
            pl.BlockSpec((1, 1, t, dh), lambda i, *_: (i // h, i % h, 0, 0)),
        ],
        out_specs=pl.BlockSpec((1, 1, t, dh), lambda i, *_: (i // h, i % h, 0, 0)),
    )
    out = pl.pallas_call(
        _attn_kernel,
        grid_spec=grid_spec,
        out_shape=jax.ShapeDtypeStruct((b, h, t, dh), q.dtype),
        compiler_params=pltpu.CompilerParams(
            dimension_semantics=("arbitrary",)),
    )(idx, w, q, k, v)
    return out


# speedup vs baseline: 1.1787x; 1.1776x over previous
import jax
import jax.numpy as jnp
from jax.experimental import pallas as pl
from jax.experimental.pallas import tpu as pltpu

_BUCKET = 128


def _router_kernel(m_ref, q_ref, k_ref, idx_ref, w_ref):
    dh, t = q_ref.shape[2], q_ref.shape[3]
    nb = t // _BUCKET
    scale = dh ** -0.5

    qT = q_ref[0, 0]
    kT = k_ref[0, 0]
    m = m_ref[...]
    sq = jax.lax.dot_general(qT, m, (((1,), (0,)), ((), ())),
                             preferred_element_type=jnp.float32)
    sk = jax.lax.dot_general(kT, m, (((1,), (0,)), ((), ())),
                             preferred_element_type=jnp.float32)
    r = jax.lax.dot_general(sq, sk, (((0,), (0,)), ((), ())),
                            preferred_element_type=jnp.float32) * scale
    rmax = jnp.max(r, axis=1, keepdims=True)
    w = 1.0 / jnp.sum(jnp.exp(r - rmax), axis=1, keepdims=True)
    iota = jax.lax.broadcasted_iota(jnp.int32, (nb, nb), 1)
    idx = jnp.min(jnp.where(r >= rmax, iota, nb), axis=1, keepdims=True)
    idx_ref[0] = idx
    w_ref[0] = w


def _attn_kernel(idx_sref, w_sref, q_ref, k_ref, v_ref, o_ref):
    dh, t = q_ref.shape[2], q_ref.shape[3]
    nb = t // _BUCKET
    scale = dh ** -0.5
    i = pl.program_id(0)

    for u in range(nb):
        idx_u = idx_sref[i, u]
        w_u = w_sref[i, u]

        qb = q_ref[0, 0, :, u * _BUCKET:(u + 1) * _BUCKET] * scale
        kl = k_ref[0, 0, :, u * _BUCKET:(u + 1) * _BUCKET]
        vl = v_ref[0, 0, :, u * _BUCKET:(u + 1) * _BUCKET]
        kg = k_ref[0, 0, :, pl.ds(idx_u * _BUCKET, _BUCKET)]
        vg = v_ref[0, 0, :, pl.ds(idx_u * _BUCKET, _BUCKET)]

        kcat = jnp.concatenate([kg * w_u, kl], axis=1)
        vcat = jnp.concatenate([vg * w_u, vl], axis=1)
        sT = jax.lax.dot_general(kcat, qb, (((0,), (0,)), ((), ())),
                                 preferred_element_type=jnp.float32)
        smax = jnp.max(sT, axis=0, keepdims=True)
        p = jnp.exp(sT - smax)
        den = jnp.sum(p, axis=0, keepdims=True)
        o = jax.lax.dot_general(vcat, p, (((1,), (0,)), ((), ())),
                                preferred_element_type=jnp.float32)
        o_ref[0, 0, :, u * _BUCKET:(u + 1) * _BUCKET] = o / den


def kernel(q, k, v):
    b, h, t, dh = q.shape
    bh = b * h
    nb = t // _BUCKET

    qT = jnp.swapaxes(q, -1, -2)
    kT = jnp.swapaxes(k, -1, -2)
    vT = jnp.swapaxes(v, -1, -2)
    mind = ((jnp.arange(t, dtype=jnp.int32)[:, None] // _BUCKET)
            == jnp.arange(nb, dtype=jnp.int32)[None, :]
            ).astype(jnp.float32) / _BUCKET

    idx3, w3 = pl.pallas_call(
        _router_kernel,
        grid=(bh,),
        in_specs=[
            pl.BlockSpec((t, nb), lambda i: (0, 0)),
            pl.BlockSpec((1, 1, dh, t), lambda i: (i // h, i % h, 0, 0)),
            pl.BlockSpec((1, 1, dh, t), lambda i: (i // h, i % h, 0, 0)),
        ],
        out_specs=[
            pl.BlockSpec((1, nb, 1), lambda i: (i, 0, 0)),
            pl.BlockSpec((1, nb, 1), lambda i: (i, 0, 0)),
        ],
        out_shape=[
            jax.ShapeDtypeStruct((bh, nb, 1), jnp.int32),
            jax.ShapeDtypeStruct((bh, nb, 1), jnp.float32),
        ],
        compiler_params=pltpu.CompilerParams(
            dimension_semantics=("arbitrary",)),
    )(mind, qT, kT)
    idx = idx3.reshape(bh, nb)
    w = w3.reshape(bh, nb)

    grid_spec = pltpu.PrefetchScalarGridSpec(
        num_scalar_prefetch=2,
        grid=(bh,),
        in_specs=[
            pl.BlockSpec((1, 1, dh, t), lambda i, *_: (i // h, i % h, 0, 0)),
            pl.BlockSpec((1, 1, dh, t), lambda i, *_: (i // h, i % h, 0, 0)),
            pl.BlockSpec((1, 1, dh, t), lambda i, *_: (i // h, i % h, 0, 0)),
        ],
        out_specs=pl.BlockSpec((1, 1, dh, t),
                               lambda i, *_: (i // h, i % h, 0, 0)),
    )
    oT = pl.pallas_call(
        _attn_kernel,
        grid_spec=grid_spec,
        out_shape=jax.ShapeDtypeStruct((b, h, dh, t), q.dtype),
        compiler_params=pltpu.CompilerParams(
            dimension_semantics=("arbitrary",)),
    )(idx, w, qT, kT, vT)
    return jnp.swapaxes(oT, -1, -2)


# speedup vs baseline: 1.3435x; 1.1398x over previous
import jax
import jax.numpy as jnp
from jax.experimental import pallas as pl
from jax.experimental.pallas import tpu as pltpu

_BUCKET = 128


def _router_kernel(m_ref, q_ref, k_ref, idx_ref, w_ref):
    dh, t = q_ref.shape[2], q_ref.shape[3]
    nb = t // _BUCKET
    scale = dh ** -0.5

    qT = q_ref[0, 0]
    kT = k_ref[0, 0]
    m = m_ref[...]
    sq = jax.lax.dot_general(qT, m, (((1,), (0,)), ((), ())),
                             preferred_element_type=jnp.float32)
    sk = jax.lax.dot_general(kT, m, (((1,), (0,)), ((), ())),
                             preferred_element_type=jnp.float32)
    r = jax.lax.dot_general(sq, sk, (((0,), (0,)), ((), ())),
                            preferred_element_type=jnp.float32) * scale
    rmax = jnp.max(r, axis=1, keepdims=True)
    w = 1.0 / jnp.sum(jnp.exp(r - rmax), axis=1, keepdims=True)
    iota = jax.lax.broadcasted_iota(jnp.int32, (nb, nb), 1)
    idx = jnp.min(jnp.where(r >= rmax, iota, nb), axis=1, keepdims=True)
    idx_ref[0] = idx
    w_ref[0] = w


def _attn_kernel(idx_sref, w_sref, q_ref, k_ref, v_ref, o_ref):
    dh, t = q_ref.shape[2], q_ref.shape[3]
    nb = t // _BUCKET
    scale = dh ** -0.5
    i = pl.program_id(0)

    for u in range(nb):
        idx_u = idx_sref[i, u]
        w_u = w_sref[i, u]

        qb = (q_ref[0, 0, :, u * _BUCKET:(u + 1) * _BUCKET]
              * scale).astype(jnp.bfloat16)
        kl = k_ref[0, 0, :, u * _BUCKET:(u + 1) * _BUCKET]
        vl = v_ref[0, 0, :, u * _BUCKET:(u + 1) * _BUCKET]
        kg = k_ref[0, 0, :, pl.ds(idx_u * _BUCKET, _BUCKET)]
        vg = v_ref[0, 0, :, pl.ds(idx_u * _BUCKET, _BUCKET)]

        kcat = jnp.concatenate([(kg * w_u).astype(jnp.bfloat16),
                                kl.astype(jnp.bfloat16)], axis=1)
        vcat = jnp.concatenate([(vg * w_u).astype(jnp.bfloat16),
                                vl.astype(jnp.bfloat16)], axis=1)
        sT = jax.lax.dot_general(kcat, qb, (((0,), (0,)), ((), ())),
                                 preferred_element_type=jnp.float32)
        p = jnp.exp(jnp.minimum(sT, 80.0))
        den = jnp.sum(p, axis=0, keepdims=True)
        o = jax.lax.dot_general(vcat, p.astype(jnp.bfloat16),
                                (((1,), (0,)), ((), ())),
                                preferred_element_type=jnp.float32)
        o_ref[0, 0, :, u * _BUCKET:(u + 1) * _BUCKET] = o / den


def kernel(q, k, v):
    b, h, t, dh = q.shape
    bh = b * h
    nb = t // _BUCKET

    qT = jnp.swapaxes(q, -1, -2)
    kT = jnp.swapaxes(k, -1, -2)
    vT = jnp.swapaxes(v, -1, -2)
    mind = ((jnp.arange(t, dtype=jnp.int32)[:, None] // _BUCKET)
            == jnp.arange(nb, dtype=jnp.int32)[None, :]
            ).astype(jnp.float32) / _BUCKET

    idx3, w3 = pl.pallas_call(
        _router_kernel,
        grid=(bh,),
        in_specs=[
            pl.BlockSpec((t, nb), lambda i: (0, 0)),
            pl.BlockSpec((1, 1, dh, t), lambda i: (i // h, i % h, 0, 0)),
            pl.BlockSpec((1, 1, dh, t), lambda i: (i // h, i % h, 0, 0)),
        ],
        out_specs=[
            pl.BlockSpec((1, nb, 1), lambda i: (i, 0, 0)),
            pl.BlockSpec((1, nb, 1), lambda i: (i, 0, 0)),
        ],
        out_shape=[
            jax.ShapeDtypeStruct((bh, nb, 1), jnp.int32),
            jax.ShapeDtypeStruct((bh, nb, 1), jnp.float32),
        ],
        compiler_params=pltpu.CompilerParams(
            dimension_semantics=("arbitrary",)),
    )(mind, qT, kT)
    idx = idx3.reshape(bh, nb)
    w = w3.reshape(bh, nb)

    grid_spec = pltpu.PrefetchScalarGridSpec(
        num_scalar_prefetch=2,
        grid=(bh,),
        in_specs=[
            pl.BlockSpec((1, 1, dh, t), lambda i, *_: (i // h, i % h, 0, 0)),
            pl.BlockSpec((1, 1, dh, t), lambda i, *_: (i // h, i % h, 0, 0)),
            pl.BlockSpec((1, 1, dh, t), lambda i, *_: (i // h, i % h, 0, 0)),
        ],
        out_specs=pl.BlockSpec((1, 1, dh, t),
                               lambda i, *_: (i // h, i % h, 0, 0)),
    )
    oT = pl.pallas_call(
        _attn_kernel,
        grid_spec=grid_spec,
        out_shape=jax.ShapeDtypeStruct((b, h, dh, t), q.dtype),
        compiler_params=pltpu.CompilerParams(
            dimension_semantics=("arbitrary",)),
    )(idx, w, qT, kT, vT)
    return jnp.swapaxes(oT, -1, -2)


# speedup vs baseline: 2.0096x; 1.4958x over previous
import jax
import jax.numpy as jnp
from jax.experimental import pallas as pl
from jax.experimental.pallas import tpu as pltpu

_BUCKET = 128


def _router_kernel(m_ref, q_ref, k_ref, idx_ref, w_ref):
    dh, t = q_ref.shape[2], q_ref.shape[3]
    nb = t // _BUCKET
    scale = dh ** -0.5

    qT = q_ref[0, 0]
    kT = k_ref[0, 0]
    m = m_ref[...]
    sq = jax.lax.dot_general(qT, m, (((1,), (0,)), ((), ())),
                             preferred_element_type=jnp.float32)
    sk = jax.lax.dot_general(kT, m, (((1,), (0,)), ((), ())),
                             preferred_element_type=jnp.float32)
    r = jax.lax.dot_general(sq, sk, (((0,), (0,)), ((), ())),
                            preferred_element_type=jnp.float32) * scale
    rmax = jnp.max(r, axis=1, keepdims=True)
    w = 1.0 / jnp.sum(jnp.exp(r - rmax), axis=1, keepdims=True)
    iota = jax.lax.broadcasted_iota(jnp.int32, (nb, nb), 1)
    idx = jnp.min(jnp.where(r >= rmax, iota, nb), axis=1, keepdims=True)
    idx_ref[0] = idx
    w_ref[0] = w


def _attn_kernel(idx_sref, w_sref, q_ref, k_ref, v_ref, o_ref):
    dh, t = q_ref.shape[2], q_ref.shape[3]
    nb = t // _BUCKET
    scale = dh ** -0.5
    i = pl.program_id(0)

    def issue_scores(u):
        idx_u = idx_sref[i, u]
        w_u = w_sref[i, u]
        qb = (q_ref[0, 0, :, u * _BUCKET:(u + 1) * _BUCKET]
              * scale).astype(jnp.bfloat16)
        kl = k_ref[0, 0, :, u * _BUCKET:(u + 1) * _BUCKET]
        vl = v_ref[0, 0, :, u * _BUCKET:(u + 1) * _BUCKET]
        kg = k_ref[0, 0, :, pl.ds(idx_u * _BUCKET, _BUCKET)]
        vg = v_ref[0, 0, :, pl.ds(idx_u * _BUCKET, _BUCKET)]
        kcat = jnp.concatenate([(kg * w_u).astype(jnp.bfloat16),
                                kl.astype(jnp.bfloat16)], axis=1)
        vcat = jnp.concatenate([(vg * w_u).astype(jnp.bfloat16),
                                vl.astype(jnp.bfloat16)], axis=1)
        sT = jax.lax.dot_general(kcat, qb, (((0,), (0,)), ((), ())),
                                 preferred_element_type=jnp.float32)
        return sT, vcat

    def finish_bucket(u, sT, vcat):
        p = jnp.exp(jnp.minimum(sT, 80.0))
        den = jnp.sum(p, axis=0, keepdims=True)
        o = jax.lax.dot_general(vcat, p.astype(jnp.bfloat16),
                                (((1,), (0,)), ((), ())),
                                preferred_element_type=jnp.float32)
        o_ref[0, 0, :, u * _BUCKET:(u + 1) * _BUCKET] = o / den

    pending = issue_scores(0)
    for u in range(1, nb):
        nxt = issue_scores(u)
        finish_bucket(u - 1, *pending)
        pending = nxt
    finish_bucket(nb - 1, *pending)


def kernel(q, k, v):
    b, h, t, dh = q.shape
    bh = b * h
    nb = t // _BUCKET

    qT = jnp.swapaxes(q, -1, -2)
    kT = jnp.swapaxes(k, -1, -2)
    vT = jnp.swapaxes(v, -1, -2)
    mind = ((jnp.arange(t, dtype=jnp.int32)[:, None] // _BUCKET)
            == jnp.arange(nb, dtype=jnp.int32)[None, :]
            ).astype(jnp.float32) / _BUCKET

    idx3, w3 = pl.pallas_call(
        _router_kernel,
        grid=(bh,),
        in_specs=[
            pl.BlockSpec((t, nb), lambda i: (0, 0)),
            pl.BlockSpec((1, 1, dh, t), lambda i: (i // h, i % h, 0, 0)),
            pl.BlockSpec((1, 1, dh, t), lambda i: (i // h, i % h, 0, 0)),
        ],
        out_specs=[
            pl.BlockSpec((1, nb, 1), lambda i: (i, 0, 0)),
            pl.BlockSpec((1, nb, 1), lambda i: (i, 0, 0)),
        ],
        out_shape=[
            jax.ShapeDtypeStruct((bh, nb, 1), jnp.int32),
            jax.ShapeDtypeStruct((bh, nb, 1), jnp.float32),
        ],
        compiler_params=pltpu.CompilerParams(
            dimension_semantics=("arbitrary",)),
    )(mind, qT, kT)
    idx = idx3.reshape(bh, nb)
    w = w3.reshape(bh, nb)

    grid_spec = pltpu.PrefetchScalarGridSpec(
        num_scalar_prefetch=2,
        grid=(bh,),
        in_specs=[
            pl.BlockSpec((1, 1, dh, t), lambda i, *_: (i // h, i % h, 0, 0)),
            pl.BlockSpec((1, 1, dh, t), lambda i, *_: (i // h, i % h, 0, 0)),
            pl.BlockSpec((1, 1, dh, t), lambda i, *_: (i // h, i % h, 0, 0)),
        ],
        out_specs=pl.BlockSpec((1, 1, dh, t),
                               lambda i, *_: (i // h, i % h, 0, 0)),
    )
    oT = pl.pallas_call(
        _attn_kernel,
        grid_spec=grid_spec,
        out_shape=jax.ShapeDtypeStruct((b, h, dh, t), q.dtype),
        compiler_params=pltpu.CompilerParams(
            dimension_semantics=("arbitrary",)),
    )(idx, w, qT, kT, vT)
    return jnp.swapaxes(oT, -1, -2)
